# baseline (speedup 1.0000x reference)
import functools

import jax
import jax.numpy as jnp
from jax import lax
from jax.experimental import pallas as pl
from jax.experimental.pallas import tpu as pltpu
from jax.experimental.pallas import tpu_sc as plsc

ROWS = 128
COLS = 32768
K = 8
L = 16
NC, NS = 2, 16
NW = NC * NS
RPW = ROWS // NW
BLK = 8
NP = 4
PW = COLS // NP
PCH = PW // L
NITER = PCH // BLK
VROWS = RPW - 1
NSEG = 4
SEGCH = PCH // NSEG

SORT8 = [(0, 1), (2, 3), (4, 5), (6, 7),
         (0, 2), (1, 3), (4, 6), (5, 7),
         (1, 2), (5, 6),
         (0, 4), (1, 5), (2, 6), (3, 7),
         (2, 4), (3, 5),
         (1, 2), (3, 4), (5, 6)]
BITONIC8 = [(0, 4), (1, 5), (2, 6), (3, 7),
            (0, 2), (1, 3), (4, 6), (5, 7),
            (0, 1), (2, 3), (4, 5), (6, 7)]

_mesh = plsc.VectorSubcoreMesh(
    core_axis_name="c", subcore_axis_name="s", num_cores=1, num_subcores=NS
)


def _merge16(best, other):
    asc, _ = plsc.sort_key_val(other, other, descending=False)
    m = jnp.maximum(best, asc)
    best, _ = plsc.sort_key_val(m, m, descending=True)
    return best


def _make_topk_sc(row_base):
  @functools.partial(
      pl.kernel,
      out_type=jax.ShapeDtypeStruct((ROWS // 2, L), jnp.float32),
      mesh=_mesh,
      scratch_types=[
          pltpu.VMEM((2, RPW, PW), jnp.float32),
          pltpu.VMEM((RPW, L), jnp.float32),
          pltpu.SemaphoreType.DMA,
          pltpu.SemaphoreType.DMA,
      ],
      compiler_params=pltpu.CompilerParams(needs_layout_passes=False),
  )
  def _topk_sc(x_hbm, out_hbm, buf, obuf, sem0, sem1):
    wid = lax.axis_index("s")
    base = row_base + wid * RPW
    obase = wid * RPW
    sems = (sem0, sem1)

    def start(q, slot):
        return [
            pltpu.async_copy(
                x_hbm.at[base + j, pl.ds(q * PW, PW)], buf.at[slot, j], sems[slot]
            )
            for j in range(RPW)
        ]

    copies = [None, None]
    copies[0] = start(0, 0)
    ninf = jnp.full((L,), -jnp.inf, dtype=jnp.float32)
    regs = (ninf,) * K
    bests = (ninf,) * (VROWS * NSEG)
    for q in range(NP):
        slot = q % 2
        if q + 1 < NP:
            copies[1 - slot] = start(q + 1, 1 - slot)
        for cp in copies[slot]:
            cp.wait()

        def body(i, carry, slot=slot):
            regs, bests = list(carry[0]), list(carry[1])
            off = i * (BLK * L)
            s = [buf[slot, 0, pl.ds(off + u * L, L)] for u in range(BLK)]
            for a, b in SORT8:
                hi = jnp.maximum(s[a], s[b])
                lo = jnp.minimum(s[a], s[b])
                s[a], s[b] = hi, lo
            c = [jnp.maximum(regs[t], s[K - 1 - t]) for t in range(K)]
            for a, b in BITONIC8:
                hi = jnp.maximum(c[a], c[b])
                lo = jnp.minimum(c[a], c[b])
                c[a], c[b] = hi, lo
            for j in range(VROWS):
                for g in range(NSEG):
                    k = j * NSEG + g
                    for t2 in range(2):
                        v = buf[slot, 1 + j, pl.ds((g * SEGCH + 2 * i + t2) * L, L)]
                        asc, _ = plsc.sort_key_val(v, v, descending=False)
                        m = jnp.maximum(bests[k], asc)
                        dsc, _ = plsc.sort_key_val(m, m, descending=True)
                        bests[k] = dsc
            return (tuple(c), tuple(bests))

        regs, bests = lax.fori_loop(0, NITER, body, (regs, bests))

    best, _ = plsc.sort_key_val(regs[0], regs[0], descending=True)
    for t in range(1, K):
        best = _merge16(best, regs[t])
    obuf[0] = best
    for j in range(VROWS):
        best = bests[j * NSEG]
        for g in range(1, NSEG):
            best = _merge16(best, bests[j * NSEG + g])
        obuf[1 + j] = best
    pltpu.sync_copy(obuf, out_hbm.at[pl.ds(obase, RPW)])

  return _topk_sc


_topk_lo = _make_topk_sc(0)
_topk_hi = _make_topk_sc(ROWS // 2)


def kernel(x):
    lo = _topk_lo(x)
    hi = _topk_hi(x)
    return jnp.concatenate([lo[:, :K], hi[:, :K]], axis=0)


# speedup vs baseline: 1.4803x; 1.4803x over previous
import functools

import jax
import jax.numpy as jnp
from jax import lax
from jax.experimental import pallas as pl
from jax.experimental.pallas import tpu as pltpu
from jax.experimental.pallas import tpu_sc as plsc

ROWS = 128
COLS = 32768
K = 8
L = 16
NC, NS = 2, 16
NW = NC * NS
RPW = ROWS // NW
BLK = 8
NP = 4
PW = COLS // NP
PCH = PW // L
NITER = PCH // BLK
VROWS = RPW - 1
NSEG = 4
SEGCH = PCH // NSEG

SORT8 = [(0, 1), (2, 3), (4, 5), (6, 7),
         (0, 2), (1, 3), (4, 6), (5, 7),
         (1, 2), (5, 6),
         (0, 4), (1, 5), (2, 6), (3, 7),
         (2, 4), (3, 5),
         (1, 2), (3, 4), (5, 6)]
BITONIC8 = [(0, 4), (1, 5), (2, 6), (3, 7),
            (0, 2), (1, 3), (4, 6), (5, 7),
            (0, 1), (2, 3), (4, 5), (6, 7)]

_mesh = plsc.VectorSubcoreMesh(
    core_axis_name="c", subcore_axis_name="s", num_cores=NC, num_subcores=NS
)


def _merge16(best, other):
    asc, _ = plsc.sort_key_val(other, other, descending=False)
    m = jnp.maximum(best, asc)
    best, _ = plsc.sort_key_val(m, m, descending=True)
    return best


@functools.partial(
    pl.kernel,
    out_type=jax.ShapeDtypeStruct((ROWS, L), jnp.float32),
    mesh=_mesh,
    scratch_types=[
        pltpu.VMEM((2, RPW, PW), jnp.float32),
        pltpu.VMEM((RPW, L), jnp.float32),
        pltpu.SemaphoreType.DMA,
        pltpu.SemaphoreType.DMA,
    ],
    compiler_params=pltpu.CompilerParams(needs_layout_passes=False),
)
def _topk_sc(x_hbm, out_hbm, buf, obuf, sem0, sem1):
    wid = lax.axis_index("s") * NC + lax.axis_index("c")
    base = wid * RPW
    sems = (sem0, sem1)

    def start(q, slot):
        return [
            pltpu.async_copy(
                x_hbm.at[base + j, pl.ds(q * PW, PW)], buf.at[slot, j], sems[slot]
            )
            for j in range(RPW)
        ]

    copies = [None, None]
    copies[0] = start(0, 0)
    ninf = jnp.full((L,), -jnp.inf, dtype=jnp.float32)
    regs = (ninf,) * K
    bests = (ninf,) * (VROWS * NSEG)
    for q in range(NP):
        slot = q % 2
        if q + 1 < NP:
            copies[1 - slot] = start(q + 1, 1 - slot)
        for cp in copies[slot]:
            cp.wait()

        def body(i, carry, slot=slot):
            regs, bests = list(carry[0]), list(carry[1])
            off = i * (BLK * L)
            s = [buf[slot, 0, pl.ds(off + u * L, L)] for u in range(BLK)]
            for a, b in SORT8:
                hi = jnp.maximum(s[a], s[b])
                lo = jnp.minimum(s[a], s[b])
                s[a], s[b] = hi, lo
            c = [jnp.maximum(regs[t], s[K - 1 - t]) for t in range(K)]
            for a, b in BITONIC8:
                hi = jnp.maximum(c[a], c[b])
                lo = jnp.minimum(c[a], c[b])
                c[a], c[b] = hi, lo
            for j in range(VROWS):
                for g in range(NSEG):
                    k = j * NSEG + g
                    for t2 in range(2):
                        v = buf[slot, 1 + j, pl.ds((g * SEGCH + 2 * i + t2) * L, L)]
                        asc, _ = plsc.sort_key_val(v, v, descending=False)
                        m = jnp.maximum(bests[k], asc)
                        dsc, _ = plsc.sort_key_val(m, m, descending=True)
                        bests[k] = dsc
            return (tuple(c), tuple(bests))

        regs, bests = lax.fori_loop(0, NITER, body, (regs, bests))

    best, _ = plsc.sort_key_val(regs[0], regs[0], descending=True)
    for t in range(1, K):
        best = _merge16(best, regs[t])
    obuf[0] = best
    for j in range(VROWS):
        best = bests[j * NSEG]
        for g in range(1, NSEG):
            best = _merge16(best, bests[j * NSEG + g])
        obuf[1 + j] = best
    pltpu.sync_copy(obuf, out_hbm.at[pl.ds(base, RPW)])


def kernel(x):
    return _topk_sc(x)[:, :K]
